# baseline (speedup 1.0000x reference)
import functools

import jax
import jax.numpy as jnp
from jax import lax
from jax.experimental import pallas as pl
from jax.experimental.pallas import tpu as pltpu
from jax.experimental.pallas import tpu_sc as plsc

N, D, G = 100000, 128, 512
B = 4000
NB = N // B

NW = 32
RT = 3136
NPAD = NW * RT
NCH = RT // 16

_NEG = -1e30


def _onehot_mask(b, n_rows):
    return b[:, None] == jax.lax.broadcasted_iota(jnp.int32, (n_rows, G), 1)


def _stage_a_kernel(x_ref, wg_ref, bg_ref, gate_ref):
    x = x_ref[...]
    gate = jax.lax.dot_general(
        x.astype(jnp.bfloat16), wg_ref[...].astype(jnp.bfloat16),
        (((1,), (0,)), ((), ())),
        preferred_element_type=jnp.float32) + bg_ref[0, 0]
    gate_ref[...] = gate.reshape(1, 1, B)


def _sc_stats_body(gate_hbm, batch_hbm, e_hbm, parts_hbm, meta_hbm,
                   gate_loc, batch_loc, e_loc, c_loc,
                   st_loc, en_loc, parts_loc, stage_loc):
    w = lax.axis_index("c") * 16 + lax.axis_index("s")
    base = w * RT
    pltpu.sync_copy(gate_hbm.at[pl.ds(base, RT)], gate_loc)
    pltpu.sync_copy(batch_hbm.at[pl.ds(base, RT)], batch_loc)

    iota = lax.iota(jnp.int32, 16)
    zi = jnp.zeros((16,), jnp.int32)
    zf = jnp.zeros((16,), jnp.float32)

    def init_chunk(k, _):
        st_loc[pl.ds(k * 16, 16)] = zi
        en_loc[pl.ds(k * 16, 16)] = zi
        parts_loc[pl.ds(k * 16, 16)] = zf
        return 0
    lax.fori_loop(0, G // 16, init_chunk, 0)

    def bdry_chunk(j, _):
        off = j * 16
        b = batch_loc[pl.ds(off, 16)]
        bp = plsc.load_gather(batch_loc, [jnp.maximum(off + iota - 1, 0)])
        is_b = b != bp
        pos = jnp.full((16,), base + off, jnp.int32) + iota
        plsc.store_scatter(st_loc, [b], pos, mask=is_b)
        plsc.store_scatter(en_loc, [bp], pos, mask=is_b)
        return 0
    lax.fori_loop(0, NCH, bdry_chunk, 0)

    lane0 = iota == 0
    b0 = batch_loc[pl.ds(0, 16)][0]
    bl = batch_loc[pl.ds(RT - 16, 16)][15]
    plsc.store_scatter(st_loc, [jnp.full((16,), b0, jnp.int32)],
                       jnp.full((16,), base, jnp.int32), mask=lane0)
    plsc.store_scatter(en_loc, [jnp.full((16,), bl, jnp.int32)],
                       jnp.full((16,), base + RT, jnp.int32), mask=lane0)

    def mx_chunk(j, carry):
        return jnp.maximum(carry, gate_loc[pl.ds(j * 16, 16)])
    tmax = jnp.max(lax.fori_loop(0, NCH, mx_chunk,
                                 jnp.full((16,), _NEG, jnp.float32)))

    def e_chunk(j, carry):
        off = j * 16
        g = gate_loc[pl.ds(off, 16)]
        e = jnp.exp(g - tmax)
        e_loc[pl.ds(off, 16)] = e
        c_loc[pl.ds(off, 16)] = plsc.cumsum(e) + carry
        return carry + jnp.sum(e)
    lax.fori_loop(0, NCH, e_chunk, jnp.float32(0.0))

    s_lo = b0
    s_hi = bl
    nch = (s_hi - s_lo + 16) // 16

    def part_chunk(k, _):
        s = jnp.full((16,), s_lo + k * 16, jnp.int32) + iota
        m = s <= s_hi
        sc = jnp.minimum(s, G - 1)
        st = plsc.load_gather(st_loc, [sc])
        en = plsc.load_gather(en_loc, [sc])
        lo_l = jnp.clip(st, base, base + RT) - base
        hi_l = jnp.clip(en, base, base + RT) - base
        vh = jnp.where(hi_l > 0,
                       plsc.load_gather(c_loc, [jnp.maximum(hi_l - 1, 0)]), 0.0)
        vl = jnp.where(lo_l > 0,
                       plsc.load_gather(c_loc, [jnp.maximum(lo_l - 1, 0)]), 0.0)
        plsc.store_scatter(parts_loc, [sc], jnp.where(m, vh - vl, 0.0), mask=m)
        return 0
    lax.fori_loop(0, nch, part_chunk, 0)

    pltpu.sync_copy(e_loc, e_hbm.at[pl.ds(base, RT)])
    pltpu.sync_copy(parts_loc, parts_hbm.at[w])

    rv = jnp.where(iota == 0, jnp.full((16,), s_lo, jnp.int32),
                   jnp.where(iota == 1, jnp.full((16,), s_hi, jnp.int32), zi))
    stage_loc[pl.ds(0, 16)] = jnp.full((16,), tmax, jnp.float32)
    stage_loc[pl.ds(16, 16)] = plsc.bitcast(rv, jnp.float32)
    pltpu.sync_copy(stage_loc, meta_hbm.at[w])


def _sc_g_body(batch_hbm, e_hbm, parts_hbm, meta_hbm, g_hbm,
               batch_loc, e_loc, g_loc, parts32_loc, rd_loc, meta_loc):
    w = lax.axis_index("c") * 16 + lax.axis_index("s")
    base = w * RT
    pltpu.sync_copy(batch_hbm.at[pl.ds(base, RT)], batch_loc)
    pltpu.sync_copy(e_hbm.at[pl.ds(base, RT)], e_loc)
    pltpu.sync_copy(parts_hbm, parts32_loc)
    pltpu.sync_copy(meta_hbm, meta_loc)

    iota = lax.iota(jnp.int32, 16)
    tms, los, his = [], [], []
    for r in range(NW):
        tms.append(meta_loc[r, pl.ds(0, 16)][0])
        rr = plsc.bitcast(meta_loc[r, pl.ds(16, 16)], jnp.int32)
        los.append(rr[0])
        his.append(rr[1])
    tm_self = meta_loc[w, pl.ds(0, 16)][0]

    def den_chunk(k, _):
        sv = jnp.full((16,), k * 16, jnp.int32) + iota
        mxs = jnp.full((16,), _NEG, jnp.float32)
        for r in range(NW):
            m = jnp.logical_and(sv >= los[r], sv <= his[r])
            mxs = jnp.where(m, jnp.maximum(mxs, jnp.full((16,), tms[r],
                                                         jnp.float32)), mxs)
        a = jnp.zeros((16,), jnp.float32)
        for r in range(NW):
            m = jnp.logical_and(sv >= los[r], sv <= his[r])
            contrib = parts32_loc[r, pl.ds(k * 16, 16)] * jnp.exp(
                jnp.full((16,), tms[r], jnp.float32) - mxs)
            a = a + jnp.where(m, contrib, 0.0)
        rd_loc[pl.ds(k * 16, 16)] = jnp.exp(
            jnp.full((16,), tm_self, jnp.float32) - mxs) / (a + 1e-16)
        return 0
    lax.fori_loop(0, G // 16, den_chunk, 0)

    def g_chunk(j, _):
        off = j * 16
        b = batch_loc[pl.ds(off, 16)]
        e = e_loc[pl.ds(off, 16)]
        g_loc[pl.ds(off, 16)] = e * plsc.load_gather(rd_loc, [b])
        return 0
    lax.fori_loop(0, NCH, g_chunk, 0)

    pltpu.sync_copy(g_loc, g_hbm.at[pl.ds(base, RT)])


def _stage_c_kernel(x_ref, g3_ref, b3_ref, out_ref, out_acc):
    i = pl.program_id(0)

    @pl.when(i == 0)
    def _():
        out_acc[...] = jnp.zeros((G, D), jnp.float32)

    mask = _onehot_mask(b3_ref[0, 0, :], B)
    g = g3_ref[...].reshape(B, 1)
    vals = x_ref[...] * g
    out_acc[...] += jax.lax.dot_general(
        mask.astype(jnp.bfloat16), vals.astype(jnp.bfloat16),
        (((0,), (0,)), ((), ())),
        preferred_element_type=jnp.float32)

    @pl.when(i == NB - 1)
    def _():
        out_ref[...] = out_acc[...]


_SC_MESH = plsc.VectorSubcoreMesh(core_axis_name="c", subcore_axis_name="s")

_sc_stats = pl.kernel(
    _sc_stats_body,
    out_type=[
        jax.ShapeDtypeStruct((NPAD,), jnp.float32),
        jax.ShapeDtypeStruct((NW, G), jnp.float32),
        jax.ShapeDtypeStruct((NW, 32), jnp.float32),
    ],
    mesh=_SC_MESH,
    scratch_types=[
        pltpu.VMEM((RT,), jnp.float32),
        pltpu.VMEM((RT,), jnp.int32),
        pltpu.VMEM((RT,), jnp.float32),
        pltpu.VMEM((RT,), jnp.float32),
        pltpu.VMEM((G,), jnp.int32),
        pltpu.VMEM((G,), jnp.int32),
        pltpu.VMEM((G,), jnp.float32),
        pltpu.VMEM((32,), jnp.float32),
    ],
    compiler_params=pltpu.CompilerParams(needs_layout_passes=False),
)

_sc_g = pl.kernel(
    _sc_g_body,
    out_type=jax.ShapeDtypeStruct((NPAD,), jnp.float32),
    mesh=_SC_MESH,
    scratch_types=[
        pltpu.VMEM((RT,), jnp.int32),
        pltpu.VMEM((RT,), jnp.float32),
        pltpu.VMEM((RT,), jnp.float32),
        pltpu.VMEM((NW, G), jnp.float32),
        pltpu.VMEM((G,), jnp.float32),
        pltpu.VMEM((NW, 32), jnp.float32),
    ],
    compiler_params=pltpu.CompilerParams(needs_layout_passes=False),
)


def kernel(x, batch, size, Wg, bg):
    del size
    bi = batch.astype(jnp.int32)
    b3 = bi.reshape(NB, 1, B)
    bg2 = bg.reshape(1, 1)

    gate3 = pl.pallas_call(
        _stage_a_kernel,
        grid=(NB,),
        in_specs=[
            pl.BlockSpec((B, D), lambda i: (i, 0)),
            pl.BlockSpec((D, 1), lambda i: (0, 0)),
            pl.BlockSpec((1, 1), lambda i: (0, 0)),
        ],
        out_specs=pl.BlockSpec((1, 1, B), lambda i: (i, 0, 0)),
        out_shape=jax.ShapeDtypeStruct((NB, 1, B), jnp.float32),
    )(x, Wg, bg2)

    gate_p = jnp.concatenate(
        [gate3.reshape(N), jnp.full((NPAD - N,), _NEG, jnp.float32)])
    batch_p = jnp.concatenate(
        [bi, jnp.full((NPAD - N,), G - 1, jnp.int32)])

    e_p, parts, meta = _sc_stats(gate_p, batch_p)
    g_p = _sc_g(batch_p, e_p, parts, meta)

    g3 = g_p[:N].reshape(NB, 1, B)
    out = pl.pallas_call(
        _stage_c_kernel,
        grid=(NB,),
        in_specs=[
            pl.BlockSpec((B, D), lambda i: (i, 0)),
            pl.BlockSpec((1, 1, B), lambda i: (i, 0, 0)),
            pl.BlockSpec((1, 1, B), lambda i: (i, 0, 0)),
        ],
        out_specs=pl.BlockSpec((G, D), lambda i: (0, 0)),
        out_shape=jax.ShapeDtypeStruct((G, D), jnp.float32),
        scratch_shapes=[pltpu.VMEM((G, D), jnp.float32)],
    )(x, g3, b3)

    g = g_p[:N].reshape(N, 1)
    return (out, g)


# speedup vs baseline: 1.1529x; 1.1529x over previous
import functools

import jax
import jax.numpy as jnp
from jax import lax
from jax.experimental import pallas as pl
from jax.experimental.pallas import tpu as pltpu
from jax.experimental.pallas import tpu_sc as plsc

N, D, G = 100000, 128, 512
B = 4000
NB = N // B

NW = 32
RT = 3136
NPAD = NW * RT
NCH = RT // 16

_NEG = -1e30


def _onehot_mask(b, n_rows):
    return b[:, None] == jax.lax.broadcasted_iota(jnp.int32, (n_rows, G), 1)


def _stage_a_kernel(x_ref, wg_ref, bg_ref, gate_ref):
    x = x_ref[...]
    w = wg_ref[...][:, 0]
    gate = jnp.sum(x * w[None, :], axis=1, keepdims=True) + bg_ref[0, 0]
    gate_ref[...] = gate.reshape(1, 1, B)


def _sc_stats_body(gate_hbm, batch_hbm, e_hbm, parts_hbm, meta_hbm,
                   gate_loc, batch_loc, e_loc, c_loc,
                   st_loc, en_loc, parts_loc, stage_loc):
    w = lax.axis_index("c") * 16 + lax.axis_index("s")
    base = w * RT
    pltpu.sync_copy(gate_hbm.at[pl.ds(base, RT)], gate_loc)
    pltpu.sync_copy(batch_hbm.at[pl.ds(base, RT)], batch_loc)

    iota = lax.iota(jnp.int32, 16)
    zi = jnp.zeros((16,), jnp.int32)
    zf = jnp.zeros((16,), jnp.float32)

    def init_chunk(k, _):
        st_loc[pl.ds(k * 16, 16)] = zi
        en_loc[pl.ds(k * 16, 16)] = zi
        parts_loc[pl.ds(k * 16, 16)] = zf
        return 0
    lax.fori_loop(0, G // 16, init_chunk, 0)

    def bdry_chunk(j, _):
        off = j * 16
        b = batch_loc[pl.ds(off, 16)]
        bp = plsc.load_gather(batch_loc, [jnp.maximum(off + iota - 1, 0)])
        is_b = b != bp
        pos = jnp.full((16,), base + off, jnp.int32) + iota
        plsc.store_scatter(st_loc, [b], pos, mask=is_b)
        plsc.store_scatter(en_loc, [bp], pos, mask=is_b)
        return 0
    lax.fori_loop(0, NCH, bdry_chunk, 0)

    lane0 = iota == 0
    b0 = batch_loc[pl.ds(0, 16)][0]
    bl = batch_loc[pl.ds(RT - 16, 16)][15]
    plsc.store_scatter(st_loc, [jnp.full((16,), b0, jnp.int32)],
                       jnp.full((16,), base, jnp.int32), mask=lane0)
    plsc.store_scatter(en_loc, [jnp.full((16,), bl, jnp.int32)],
                       jnp.full((16,), base + RT, jnp.int32), mask=lane0)

    def mx_chunk(j, carry):
        return jnp.maximum(carry, gate_loc[pl.ds(j * 16, 16)])
    tmax = jnp.max(lax.fori_loop(0, NCH, mx_chunk,
                                 jnp.full((16,), _NEG, jnp.float32)))

    def e_chunk(j, carry):
        off = j * 16
        g = gate_loc[pl.ds(off, 16)]
        e = jnp.exp(g - tmax)
        e_loc[pl.ds(off, 16)] = e
        c_loc[pl.ds(off, 16)] = plsc.cumsum(e) + carry
        return carry + jnp.sum(e)
    lax.fori_loop(0, NCH, e_chunk, jnp.float32(0.0))

    s_lo = b0
    s_hi = bl
    nch = (s_hi - s_lo + 16) // 16

    def part_chunk(k, _):
        s = jnp.full((16,), s_lo + k * 16, jnp.int32) + iota
        m = s <= s_hi
        sc = jnp.minimum(s, G - 1)
        st = plsc.load_gather(st_loc, [sc])
        en = plsc.load_gather(en_loc, [sc])
        lo_l = jnp.clip(st, base, base + RT) - base
        hi_l = jnp.clip(en, base, base + RT) - base
        vh = jnp.where(hi_l > 0,
                       plsc.load_gather(c_loc, [jnp.maximum(hi_l - 1, 0)]), 0.0)
        vl = jnp.where(lo_l > 0,
                       plsc.load_gather(c_loc, [jnp.maximum(lo_l - 1, 0)]), 0.0)
        plsc.store_scatter(parts_loc, [sc], jnp.where(m, vh - vl, 0.0), mask=m)
        return 0
    lax.fori_loop(0, nch, part_chunk, 0)

    pltpu.sync_copy(e_loc, e_hbm.at[pl.ds(base, RT)])
    pltpu.sync_copy(parts_loc, parts_hbm.at[w])

    rv = jnp.where(iota == 0, jnp.full((16,), s_lo, jnp.int32),
                   jnp.where(iota == 1, jnp.full((16,), s_hi, jnp.int32), zi))
    stage_loc[pl.ds(0, 16)] = jnp.full((16,), tmax, jnp.float32)
    stage_loc[pl.ds(16, 16)] = plsc.bitcast(rv, jnp.float32)
    pltpu.sync_copy(stage_loc, meta_hbm.at[w])


def _sc_g_body(batch_hbm, e_hbm, parts_hbm, meta_hbm, g_hbm,
               batch_loc, e_loc, g_loc, parts32_loc, rd_loc, meta_loc):
    w = lax.axis_index("c") * 16 + lax.axis_index("s")
    base = w * RT
    pltpu.sync_copy(batch_hbm.at[pl.ds(base, RT)], batch_loc)
    pltpu.sync_copy(e_hbm.at[pl.ds(base, RT)], e_loc)
    pltpu.sync_copy(parts_hbm, parts32_loc)
    pltpu.sync_copy(meta_hbm, meta_loc)

    iota = lax.iota(jnp.int32, 16)
    tms, los, his = [], [], []
    for r in range(NW):
        tms.append(meta_loc[r, pl.ds(0, 16)][0])
        rr = plsc.bitcast(meta_loc[r, pl.ds(16, 16)], jnp.int32)
        los.append(rr[0])
        his.append(rr[1])
    tm_self = meta_loc[w, pl.ds(0, 16)][0]

    def den_chunk(k, _):
        sv = jnp.full((16,), k * 16, jnp.int32) + iota
        mxs = jnp.full((16,), _NEG, jnp.float32)
        for r in range(NW):
            m = jnp.logical_and(sv >= los[r], sv <= his[r])
            mxs = jnp.where(m, jnp.maximum(mxs, jnp.full((16,), tms[r],
                                                         jnp.float32)), mxs)
        a = jnp.zeros((16,), jnp.float32)
        for r in range(NW):
            m = jnp.logical_and(sv >= los[r], sv <= his[r])
            contrib = parts32_loc[r, pl.ds(k * 16, 16)] * jnp.exp(
                jnp.full((16,), tms[r], jnp.float32) - mxs)
            a = a + jnp.where(m, contrib, 0.0)
        rd_loc[pl.ds(k * 16, 16)] = jnp.exp(
            jnp.full((16,), tm_self, jnp.float32) - mxs) / (a + 1e-16)
        return 0
    lax.fori_loop(0, G // 16, den_chunk, 0)

    def g_chunk(j, _):
        off = j * 16
        b = batch_loc[pl.ds(off, 16)]
        e = e_loc[pl.ds(off, 16)]
        g_loc[pl.ds(off, 16)] = e * plsc.load_gather(rd_loc, [b])
        return 0
    lax.fori_loop(0, NCH, g_chunk, 0)

    pltpu.sync_copy(g_loc, g_hbm.at[pl.ds(base, RT)])


WIN = 128
GPAD = G + WIN


def _stage_c_kernel(x_ref, g3_ref, b3_ref, out_ref, out_acc):
    i = pl.program_id(0)

    @pl.when(i == 0)
    def _():
        out_acc[...] = jnp.zeros((GPAD, D), jnp.float32)

    b = b3_ref[0, 0, :]
    g = g3_ref[...].reshape(B, 1)
    vals = (x_ref[...] * g).astype(jnp.bfloat16)

    s0a = (b[0] // 8) * 8
    nwin = (b[B - 1] - s0a) // WIN + 1

    def win(t, _):
        base_s = pl.multiple_of(s0a + t * WIN, 8)
        mask = (b[:, None] - base_s) == jax.lax.broadcasted_iota(
            jnp.int32, (B, WIN), 1)
        part = jax.lax.dot_general(
            mask.astype(jnp.bfloat16), vals,
            (((0,), (0,)), ((), ())),
            preferred_element_type=jnp.float32)
        out_acc[pl.ds(base_s, WIN), :] += part
        return 0
    jax.lax.fori_loop(0, nwin, win, 0)

    @pl.when(i == NB - 1)
    def _():
        out_ref[...] = out_acc[pl.ds(0, G), :]


_SC_MESH = plsc.VectorSubcoreMesh(core_axis_name="c", subcore_axis_name="s")

_sc_stats = pl.kernel(
    _sc_stats_body,
    out_type=[
        jax.ShapeDtypeStruct((NPAD,), jnp.float32),
        jax.ShapeDtypeStruct((NW, G), jnp.float32),
        jax.ShapeDtypeStruct((NW, 32), jnp.float32),
    ],
    mesh=_SC_MESH,
    scratch_types=[
        pltpu.VMEM((RT,), jnp.float32),
        pltpu.VMEM((RT,), jnp.int32),
        pltpu.VMEM((RT,), jnp.float32),
        pltpu.VMEM((RT,), jnp.float32),
        pltpu.VMEM((G,), jnp.int32),
        pltpu.VMEM((G,), jnp.int32),
        pltpu.VMEM((G,), jnp.float32),
        pltpu.VMEM((32,), jnp.float32),
    ],
    compiler_params=pltpu.CompilerParams(needs_layout_passes=False),
)

_sc_g = pl.kernel(
    _sc_g_body,
    out_type=jax.ShapeDtypeStruct((NPAD,), jnp.float32),
    mesh=_SC_MESH,
    scratch_types=[
        pltpu.VMEM((RT,), jnp.int32),
        pltpu.VMEM((RT,), jnp.float32),
        pltpu.VMEM((RT,), jnp.float32),
        pltpu.VMEM((NW, G), jnp.float32),
        pltpu.VMEM((G,), jnp.float32),
        pltpu.VMEM((NW, 32), jnp.float32),
    ],
    compiler_params=pltpu.CompilerParams(needs_layout_passes=False),
)


def kernel(x, batch, size, Wg, bg):
    del size
    bi = batch.astype(jnp.int32)
    b3 = bi.reshape(NB, 1, B)
    bg2 = bg.reshape(1, 1)

    gate3 = pl.pallas_call(
        _stage_a_kernel,
        grid=(NB,),
        in_specs=[
            pl.BlockSpec((B, D), lambda i: (i, 0)),
            pl.BlockSpec((D, 1), lambda i: (0, 0)),
            pl.BlockSpec((1, 1), lambda i: (0, 0)),
        ],
        out_specs=pl.BlockSpec((1, 1, B), lambda i: (i, 0, 0)),
        out_shape=jax.ShapeDtypeStruct((NB, 1, B), jnp.float32),
    )(x, Wg, bg2)

    gate_p = jnp.concatenate(
        [gate3.reshape(N), jnp.full((NPAD - N,), _NEG, jnp.float32)])
    batch_p = jnp.concatenate(
        [bi, jnp.full((NPAD - N,), G - 1, jnp.int32)])

    e_p, parts, meta = _sc_stats(gate_p, batch_p)
    g_p = _sc_g(batch_p, e_p, parts, meta)

    g3 = g_p[:N].reshape(NB, 1, B)
    out = pl.pallas_call(
        _stage_c_kernel,
        grid=(NB,),
        in_specs=[
            pl.BlockSpec((B, D), lambda i: (i, 0)),
            pl.BlockSpec((1, 1, B), lambda i: (i, 0, 0)),
            pl.BlockSpec((1, 1, B), lambda i: (i, 0, 0)),
        ],
        out_specs=pl.BlockSpec((G, D), lambda i: (0, 0)),
        out_shape=jax.ShapeDtypeStruct((G, D), jnp.float32),
        scratch_shapes=[pltpu.VMEM((GPAD, D), jnp.float32)],
    )(x, g3, b3)

    g = g_p[:N].reshape(N, 1)
    return (out, g)


# speedup vs baseline: 1.1668x; 1.0120x over previous
import functools

import jax
import jax.numpy as jnp
from jax import lax
from jax.experimental import pallas as pl
from jax.experimental.pallas import tpu as pltpu
from jax.experimental.pallas import tpu_sc as plsc

N, D, G = 100000, 128, 512
B = 4000
NB = N // B

NW = 32
RT = 3136
NPAD = NW * RT
NCH = RT // 16

_NEG = -1e30


def _onehot_mask(b, n_rows):
    return b[:, None] == jax.lax.broadcasted_iota(jnp.int32, (n_rows, G), 1)


def _stage_a_kernel(x_ref, wg_ref, bg_ref, gate_ref):
    x = x_ref[...]
    w = wg_ref[...][:, 0]
    gate = jnp.sum(x * w[None, :], axis=1, keepdims=True) + bg_ref[0, 0]
    gate_ref[...] = gate.reshape(1, 1, B)


def _sc_stats_body(gate_hbm, batch_hbm, e_hbm, parts_hbm, meta_hbm,
                   gate_loc, batch_loc, e_loc, c_loc,
                   st_loc, en_loc, parts_loc, stage_loc):
    w = lax.axis_index("c") * 16 + lax.axis_index("s")
    base = w * RT
    pltpu.sync_copy(gate_hbm.at[pl.ds(base, RT)], gate_loc)
    pltpu.sync_copy(batch_hbm.at[pl.ds(base, RT)], batch_loc)

    iota = lax.iota(jnp.int32, 16)
    zi = jnp.zeros((16,), jnp.int32)
    zf = jnp.zeros((16,), jnp.float32)

    def init_chunk(k, _):
        st_loc[pl.ds(k * 16, 16)] = zi
        en_loc[pl.ds(k * 16, 16)] = zi
        parts_loc[pl.ds(k * 16, 16)] = zf
        return 0
    lax.fori_loop(0, G // 16, init_chunk, 0)

    def bdry_chunk(j, _):
        off = j * 16
        b = batch_loc[pl.ds(off, 16)]
        bp = plsc.load_gather(batch_loc, [jnp.maximum(off + iota - 1, 0)])
        is_b = b != bp
        pos = jnp.full((16,), base + off, jnp.int32) + iota
        plsc.store_scatter(st_loc, [b], pos, mask=is_b)
        plsc.store_scatter(en_loc, [bp], pos, mask=is_b)
        return 0
    lax.fori_loop(0, NCH, bdry_chunk, 0)

    lane0 = iota == 0
    b0 = batch_loc[pl.ds(0, 16)][0]
    bl = batch_loc[pl.ds(RT - 16, 16)][15]
    plsc.store_scatter(st_loc, [jnp.full((16,), b0, jnp.int32)],
                       jnp.full((16,), base, jnp.int32), mask=lane0)
    plsc.store_scatter(en_loc, [jnp.full((16,), bl, jnp.int32)],
                       jnp.full((16,), base + RT, jnp.int32), mask=lane0)

    def mx_chunk(j, carry):
        return jnp.maximum(carry, gate_loc[pl.ds(j * 16, 16)])
    tmax = jnp.max(lax.fori_loop(0, NCH, mx_chunk,
                                 jnp.full((16,), _NEG, jnp.float32)))

    def e_chunk(j, carry):
        off = j * 16
        g = gate_loc[pl.ds(off, 16)]
        e = jnp.exp(g - tmax)
        e_loc[pl.ds(off, 16)] = e
        c_loc[pl.ds(off, 16)] = plsc.cumsum(e) + carry
        return carry + jnp.sum(e)
    lax.fori_loop(0, NCH, e_chunk, jnp.float32(0.0))

    s_lo = b0
    s_hi = bl
    nch = (s_hi - s_lo + 16) // 16

    def part_chunk(k, _):
        s = jnp.full((16,), s_lo + k * 16, jnp.int32) + iota
        m = s <= s_hi
        sc = jnp.minimum(s, G - 1)
        st = plsc.load_gather(st_loc, [sc])
        en = plsc.load_gather(en_loc, [sc])
        lo_l = jnp.clip(st, base, base + RT) - base
        hi_l = jnp.clip(en, base, base + RT) - base
        vh = jnp.where(hi_l > 0,
                       plsc.load_gather(c_loc, [jnp.maximum(hi_l - 1, 0)]), 0.0)
        vl = jnp.where(lo_l > 0,
                       plsc.load_gather(c_loc, [jnp.maximum(lo_l - 1, 0)]), 0.0)
        plsc.store_scatter(parts_loc, [sc], jnp.where(m, vh - vl, 0.0), mask=m)
        return 0
    lax.fori_loop(0, nch, part_chunk, 0)

    pltpu.sync_copy(e_loc, e_hbm.at[pl.ds(base, RT)])
    pltpu.sync_copy(parts_loc, parts_hbm.at[w])

    rv = jnp.where(iota == 0, jnp.full((16,), s_lo, jnp.int32),
                   jnp.where(iota == 1, jnp.full((16,), s_hi, jnp.int32), zi))
    stage_loc[pl.ds(0, 16)] = jnp.full((16,), tmax, jnp.float32)
    stage_loc[pl.ds(16, 16)] = plsc.bitcast(rv, jnp.float32)
    pltpu.sync_copy(stage_loc, meta_hbm.at[w])


def _sc_g_body(batch_hbm, e_hbm, parts_hbm, meta_hbm, g_hbm,
               batch_loc, e_loc, g_loc, parts32_loc, rd_loc, meta_loc):
    w = lax.axis_index("c") * 16 + lax.axis_index("s")
    base = w * RT
    pltpu.sync_copy(batch_hbm.at[pl.ds(base, RT)], batch_loc)
    pltpu.sync_copy(e_hbm.at[pl.ds(base, RT)], e_loc)
    pltpu.sync_copy(parts_hbm, parts32_loc)
    pltpu.sync_copy(meta_hbm, meta_loc)

    iota = lax.iota(jnp.int32, 16)
    tms, los, his = [], [], []
    for r in range(NW):
        tms.append(meta_loc[r, pl.ds(0, 16)][0])
        rr = plsc.bitcast(meta_loc[r, pl.ds(16, 16)], jnp.int32)
        los.append(rr[0])
        his.append(rr[1])
    tm_self = meta_loc[w, pl.ds(0, 16)][0]

    def den_chunk(k, _):
        sv = jnp.full((16,), k * 16, jnp.int32) + iota
        mxs = jnp.full((16,), _NEG, jnp.float32)
        for r in range(NW):
            m = jnp.logical_and(sv >= los[r], sv <= his[r])
            mxs = jnp.where(m, jnp.maximum(mxs, jnp.full((16,), tms[r],
                                                         jnp.float32)), mxs)
        a = jnp.zeros((16,), jnp.float32)
        for r in range(NW):
            m = jnp.logical_and(sv >= los[r], sv <= his[r])
            contrib = parts32_loc[r, pl.ds(k * 16, 16)] * jnp.exp(
                jnp.full((16,), tms[r], jnp.float32) - mxs)
            a = a + jnp.where(m, contrib, 0.0)
        rd_loc[pl.ds(k * 16, 16)] = jnp.exp(
            jnp.full((16,), tm_self, jnp.float32) - mxs) / (a + 1e-16)
        return 0
    lax.fori_loop(0, G // 16, den_chunk, 0)

    def g_chunk(j, _):
        off = j * 16
        b = batch_loc[pl.ds(off, 16)]
        e = e_loc[pl.ds(off, 16)]
        g_loc[pl.ds(off, 16)] = e * plsc.load_gather(rd_loc, [b])
        return 0
    lax.fori_loop(0, NCH, g_chunk, 0)

    pltpu.sync_copy(g_loc, g_hbm.at[pl.ds(base, RT)])


WIN = 64
GPAD = G + WIN


def _stage_c_kernel(x_ref, g3_ref, b3_ref, out_ref, out_acc):
    i = pl.program_id(0)

    @pl.when(i == 0)
    def _():
        out_acc[...] = jnp.zeros((GPAD, D), jnp.float32)

    b = b3_ref[0, 0, :]
    g = g3_ref[...].reshape(B, 1)
    vals = (x_ref[...] * g).astype(jnp.bfloat16)

    s0a = (b[0] // 8) * 8
    nwin = (b[B - 1] - s0a) // WIN + 1

    def win(t, _):
        base_s = pl.multiple_of(s0a + t * WIN, 8)
        mask = (b[:, None] - base_s) == jax.lax.broadcasted_iota(
            jnp.int32, (B, WIN), 1)
        part = jax.lax.dot_general(
            mask.astype(jnp.bfloat16), vals,
            (((0,), (0,)), ((), ())),
            preferred_element_type=jnp.float32)
        out_acc[pl.ds(base_s, WIN), :] += part
        return 0
    jax.lax.fori_loop(0, nwin, win, 0)

    @pl.when(i == NB - 1)
    def _():
        out_ref[...] = out_acc[pl.ds(0, G), :]


_SC_MESH = plsc.VectorSubcoreMesh(core_axis_name="c", subcore_axis_name="s")

_sc_stats = pl.kernel(
    _sc_stats_body,
    out_type=[
        jax.ShapeDtypeStruct((NPAD,), jnp.float32),
        jax.ShapeDtypeStruct((NW, G), jnp.float32),
        jax.ShapeDtypeStruct((NW, 32), jnp.float32),
    ],
    mesh=_SC_MESH,
    scratch_types=[
        pltpu.VMEM((RT,), jnp.float32),
        pltpu.VMEM((RT,), jnp.int32),
        pltpu.VMEM((RT,), jnp.float32),
        pltpu.VMEM((RT,), jnp.float32),
        pltpu.VMEM((G,), jnp.int32),
        pltpu.VMEM((G,), jnp.int32),
        pltpu.VMEM((G,), jnp.float32),
        pltpu.VMEM((32,), jnp.float32),
    ],
    compiler_params=pltpu.CompilerParams(needs_layout_passes=False),
)

_sc_g = pl.kernel(
    _sc_g_body,
    out_type=jax.ShapeDtypeStruct((NPAD,), jnp.float32),
    mesh=_SC_MESH,
    scratch_types=[
        pltpu.VMEM((RT,), jnp.int32),
        pltpu.VMEM((RT,), jnp.float32),
        pltpu.VMEM((RT,), jnp.float32),
        pltpu.VMEM((NW, G), jnp.float32),
        pltpu.VMEM((G,), jnp.float32),
        pltpu.VMEM((NW, 32), jnp.float32),
    ],
    compiler_params=pltpu.CompilerParams(needs_layout_passes=False),
)


def kernel(x, batch, size, Wg, bg):
    del size
    bi = batch.astype(jnp.int32)
    b3 = bi.reshape(NB, 1, B)
    bg2 = bg.reshape(1, 1)

    gate3 = pl.pallas_call(
        _stage_a_kernel,
        grid=(NB,),
        in_specs=[
            pl.BlockSpec((B, D), lambda i: (i, 0)),
            pl.BlockSpec((D, 1), lambda i: (0, 0)),
            pl.BlockSpec((1, 1), lambda i: (0, 0)),
        ],
        out_specs=pl.BlockSpec((1, 1, B), lambda i: (i, 0, 0)),
        out_shape=jax.ShapeDtypeStruct((NB, 1, B), jnp.float32),
    )(x, Wg, bg2)

    gate_p = jnp.concatenate(
        [gate3.reshape(N), jnp.full((NPAD - N,), _NEG, jnp.float32)])
    batch_p = jnp.concatenate(
        [bi, jnp.full((NPAD - N,), G - 1, jnp.int32)])

    e_p, parts, meta = _sc_stats(gate_p, batch_p)
    g_p = _sc_g(batch_p, e_p, parts, meta)

    g3 = g_p[:N].reshape(NB, 1, B)
    out = pl.pallas_call(
        _stage_c_kernel,
        grid=(NB,),
        in_specs=[
            pl.BlockSpec((B, D), lambda i: (i, 0)),
            pl.BlockSpec((1, 1, B), lambda i: (i, 0, 0)),
            pl.BlockSpec((1, 1, B), lambda i: (i, 0, 0)),
        ],
        out_specs=pl.BlockSpec((G, D), lambda i: (0, 0)),
        out_shape=jax.ShapeDtypeStruct((G, D), jnp.float32),
        scratch_shapes=[pltpu.VMEM((GPAD, D), jnp.float32)],
    )(x, g3, b3)

    g = g_p[:N].reshape(N, 1)
    return (out, g)
